# baseline (speedup 1.0000x reference)
import jax
import jax.numpy as jnp
from jax.experimental import pallas as pl
from jax.experimental.pallas import tpu as pltpu

_TOKENS = 8192
_DIM = 2048
_EXPERTS = 64
_BT = 512


def _gate_kernel(x_ref, w_ref, out_ref):
    logits = jax.lax.dot_general(
        x_ref[...], w_ref[...],
        dimension_numbers=(((1,), (1,)), ((), ())),
        preferred_element_type=jnp.float32)
    m = jnp.max(logits, axis=1, keepdims=True)
    e = jnp.exp(logits - m)
    out_ref[...] = e / jnp.sum(e, axis=1, keepdims=True)


def kernel(inputs, wg):
    return pl.pallas_call(
        _gate_kernel,
        grid=(_TOKENS // _BT,),
        in_specs=[
            pl.BlockSpec((_BT, _DIM), lambda i: (i, 0)),
            pl.BlockSpec((_EXPERTS, _DIM), lambda i: (0, 0)),
        ],
        out_specs=pl.BlockSpec((_BT, _EXPERTS), lambda i: (i, 0)),
        out_shape=jax.ShapeDtypeStruct((_TOKENS, _EXPERTS), jnp.float32),
        compiler_params=pltpu.CompilerParams(
            dimension_semantics=("parallel",)),
    )(inputs, wg)


# speedup vs baseline: 1.0603x; 1.0603x over previous
import jax
import jax.numpy as jnp
from jax.experimental import pallas as pl
from jax.experimental.pallas import tpu as pltpu

_TOKENS = 8192
_DIM = 2048
_EXPERTS = 64
_BT = 1024


def _gate_kernel(x_ref, w_hbm, out_ref, w_vmem, w_sem):
    @pl.when(pl.program_id(0) == 0)
    def _():
        pltpu.make_async_copy(w_hbm, w_vmem, w_sem).start()
        pltpu.make_async_copy(w_hbm, w_vmem, w_sem).wait()

    logits = jax.lax.dot_general(
        x_ref[...], w_vmem[...],
        dimension_numbers=(((1,), (1,)), ((), ())),
        preferred_element_type=jnp.float32)
    m = jnp.max(logits, axis=1, keepdims=True)
    e = jnp.exp(logits - m)
    out_ref[...] = e / jnp.sum(e, axis=1, keepdims=True)


def kernel(inputs, wg):
    return pl.pallas_call(
        _gate_kernel,
        grid=(_TOKENS // _BT,),
        in_specs=[
            pl.BlockSpec((_BT, _DIM), lambda i: (i, 0)),
            pl.BlockSpec(memory_space=pltpu.MemorySpace.HBM),
        ],
        out_specs=pl.BlockSpec((_BT, _EXPERTS), lambda i: (i, 0)),
        out_shape=jax.ShapeDtypeStruct((_TOKENS, _EXPERTS), jnp.float32),
        scratch_shapes=[
            pltpu.VMEM((_EXPERTS, _DIM), jnp.float32),
            pltpu.SemaphoreType.DMA,
        ],
        compiler_params=pltpu.CompilerParams(
            dimension_semantics=("arbitrary",)),
    )(inputs, wg)


# speedup vs baseline: 1.1210x; 1.0573x over previous
import jax
import jax.numpy as jnp
from jax.experimental import pallas as pl
from jax.experimental.pallas import tpu as pltpu

_TOKENS = 8192
_DIM = 2048
_EXPERTS = 64
_BT = 1024


_NSTEP = _TOKENS // _BT


def _gate_kernel(x_ref, w_ref, out_hbm, obuf, out_sems):
    i = pl.program_id(0)
    slot = jax.lax.rem(i, 2)

    logits = jax.lax.dot_general(
        x_ref[...], w_ref[...],
        dimension_numbers=(((1,), (1,)), ((), ())),
        preferred_element_type=jnp.float32)
    m = jnp.max(logits, axis=1, keepdims=True)
    e = jnp.exp(logits - m)

    def st_copy(step, s):
        return pltpu.make_async_copy(
            obuf.at[s], out_hbm.at[pl.ds(step * _BT, _BT), :], out_sems.at[s])

    @pl.when(i >= 2)
    def _():
        st_copy(i - 2, slot).wait()

    obuf[slot] = e / jnp.sum(e, axis=1, keepdims=True)
    st_copy(i, slot).start()

    @pl.when(i == _NSTEP - 1)
    def _():
        st_copy(i - 1, 1 - slot).wait()
        st_copy(i, slot).wait()


def kernel(inputs, wg):
    return pl.pallas_call(
        _gate_kernel,
        grid=(_NSTEP,),
        in_specs=[
            pl.BlockSpec((_BT, _DIM), lambda i: (i, 0)),
            pl.BlockSpec((_EXPERTS, _DIM), lambda i: (0, 0)),
        ],
        out_specs=pl.BlockSpec(memory_space=pltpu.MemorySpace.HBM),
        out_shape=jax.ShapeDtypeStruct((_TOKENS, _EXPERTS), jnp.float32),
        scratch_shapes=[
            pltpu.VMEM((2, _BT, _EXPERTS), jnp.float32),
            pltpu.SemaphoreType.DMA((2,)),
        ],
        compiler_params=pltpu.CompilerParams(
            dimension_semantics=("arbitrary",)),
    )(inputs, wg)
